# Initial kernel scaffold; baseline (speedup 1.0000x reference)
import jax
import jax.numpy as jnp
from jax.experimental import pallas as pl


def kernel(item_id, hist_flat, cu_seqlens, item_table, att_W1, att_b1, att_W2, att_b2, fc_W1, fc_b1, fc_W2, fc_b2, fc_W3, fc_b3):
    raise NotImplementedError("write your pallas kernel here")


# speedup vs baseline: 78.4010x; 78.4010x over previous
import functools

import jax
import jax.numpy as jnp
from jax import lax
from jax.experimental import pallas as pl
from jax.experimental.pallas import tpu as pltpu
from jax.experimental.pallas import tpu_sc as plsc

NC = 2
NS = 16
NW = NC * NS
CH = 128
SBK = 1024
SB = 128
TILE = 1024


def _mesh():
    return plsc.VectorSubcoreMesh(
        core_axis_name="c", subcore_axis_name="s", num_cores=NC, num_subcores=NS
    )


@functools.lru_cache(maxsize=None)
def _sc_gather(T, B, D):
    TPW = T // NW
    NSB = TPW // SBK
    NGH = SBK // CH
    BPW = B // NW
    assert NSB % 2 == 0

    def body(hist_flat, item_id, table, hist_o, tgt_o,
             idx_a, idx_b, rows_a, rows_b, gsem_a, gsem_b, wsem_a, wsem_b):
        c = lax.axis_index("c")
        s = lax.axis_index("s")
        wid = s * NC + c
        base = wid * TPW
        idx = (idx_a, idx_b)
        rows = (rows_a, rows_b)
        gsem = (gsem_a, gsem_b)
        wsem = (wsem_a, wsem_b)

        def fire_gathers(sb, slot):
            start = base + sb * SBK
            pltpu.sync_copy(hist_flat.at[pl.ds(start, SBK)], idx[slot])
            for k in range(NGH):
                pltpu.async_copy(
                    table.at[idx[slot].at[pl.ds(k * CH, CH)]],
                    rows[slot].at[pl.ds(k * CH, CH)],
                    gsem[slot],
                )

        def drain_gathers(slot):
            pltpu.make_async_copy(
                hist_o.at[pl.ds(0, SBK)], rows[slot], gsem[slot]
            ).wait()

        def fire_write(sb, slot):
            start = base + sb * SBK
            pltpu.async_copy(rows[slot], hist_o.at[pl.ds(start, SBK)], wsem[slot])

        def drain_write(slot):
            pltpu.make_async_copy(
                rows[slot], hist_o.at[pl.ds(0, SBK)], wsem[slot]
            ).wait()

        fire_gathers(0, 0)

        def pair(p, carry):
            for phase in range(2):
                sb = 2 * p + phase
                slot = phase
                other = 1 - phase
                drain_gathers(slot)

                @pl.when(sb > 0)
                def _():
                    drain_write(other)

                @pl.when(sb + 1 < NSB)
                def _():
                    fire_gathers(sb + 1, other)

                fire_write(sb, slot)
            return carry

        lax.fori_loop(0, NSB // 2, pair, 0)
        drain_write(1)

        bstart = wid * BPW
        pltpu.sync_copy(item_id.at[pl.ds(bstart, BPW)], idx_a.at[pl.ds(0, BPW)])
        for k in range(BPW // CH):
            pltpu.async_copy(
                table.at[idx_a.at[pl.ds(k * CH, CH)]],
                rows_a.at[pl.ds(k * CH, CH)],
                gsem_a,
            )
        pltpu.make_async_copy(
            hist_o.at[pl.ds(0, BPW)], rows_a.at[pl.ds(0, BPW)], gsem_a
        ).wait()
        pltpu.sync_copy(rows_a.at[pl.ds(0, BPW)], tgt_o.at[pl.ds(bstart, BPW)])

    return pl.kernel(
        body,
        out_type=(
            jax.ShapeDtypeStruct((T, D), jnp.float32),
            jax.ShapeDtypeStruct((B, D), jnp.float32),
        ),
        mesh=_mesh(),
        scratch_types=(
            pltpu.VMEM((SBK,), jnp.int32),
            pltpu.VMEM((SBK,), jnp.int32),
            pltpu.VMEM((SBK, D), jnp.float32),
            pltpu.VMEM((SBK, D), jnp.float32),
            pltpu.SemaphoreType.DMA,
            pltpu.SemaphoreType.DMA,
            pltpu.SemaphoreType.DMA,
            pltpu.SemaphoreType.DMA,
        ),
        compiler_params=pltpu.CompilerParams(
            needs_layout_passes=False, use_tc_tiling_on_sc=False
        ),
    )


@functools.lru_cache(maxsize=None)
def _tc_din(T, B, D, H1, H2):
    G = B // SB

    def body(cu_sp, cub_ref, tgt_ref, w1_ref, b1_ref, w2_ref, b2_ref,
             fw1_ref, fb1_ref, fw2_ref, fb2_ref, fw3_ref, fb3_ref,
             hist_hbm, out_ref, acc_ref):
        g = pl.program_id(0)
        s_begin = cu_sp[g * SB]
        s_end = cu_sp[g * SB + SB]
        tile0 = s_begin // TILE
        ntiles = lax.div(s_end + (TILE - 1), TILE) - tile0

        acc_ref[...] = jnp.zeros_like(acc_ref)
        lo = cub_ref[0, 0, 0:SB]
        hi = cub_ref[0, 0, 1:SB + 1]
        lo_b = jnp.broadcast_to(lo.reshape(1, SB), (TILE, SB))
        hi_b = jnp.broadcast_to(hi.reshape(1, SB), (TILE, SB))
        tgt_blk = tgt_ref[...]

        def tile_body(idxs, e_ref):
            (k,) = idxs
            tok0 = (tile0 + k) * TILE
            e = e_ref[...]
            tpos = tok0 + lax.broadcasted_iota(jnp.int32, (TILE, SB), 0)
            mf = jnp.where((tpos >= lo_b) & (tpos < hi_b), 1.0, 0.0)
            tb = jnp.dot(mf, tgt_blk, preferred_element_type=jnp.float32)
            feat = jnp.concatenate([e, tb, e * tb], axis=1)
            z = jnp.dot(feat, w1_ref[...], preferred_element_type=jnp.float32)
            z = jnp.maximum(z + b1_ref[...], 0.0)
            sc = jnp.dot(z, w2_ref[...], preferred_element_type=jnp.float32)
            w = jnp.exp(sc + b2_ref[0, 0])
            we = jnp.concatenate([w * e, w], axis=1)
            acc_ref[...] += lax.dot_general(
                mf, we, (((0,), (0,)), ((), ())),
                preferred_element_type=jnp.float32,
            )

        pipe = pltpu.emit_pipeline(
            tile_body,
            grid=(ntiles,),
            in_specs=[pl.BlockSpec((TILE, D), lambda k: (tile0 + k, 0))],
            _explicit_indices=True,
        )
        pipe(hist_hbm)

        a = acc_ref[...]
        weighted = a[:, 0:D] / (a[:, D:D + 1] + 1e-9)
        comb = jnp.concatenate([tgt_blk, weighted], axis=1)
        h1 = jnp.maximum(
            jnp.dot(comb, fw1_ref[...], preferred_element_type=jnp.float32)
            + fb1_ref[...], 0.0)
        h2 = jnp.maximum(
            jnp.dot(h1, fw2_ref[...], preferred_element_type=jnp.float32)
            + fb2_ref[...], 0.0)
        o = jax.nn.sigmoid(
            jnp.dot(h2, fw3_ref[...], preferred_element_type=jnp.float32)
            + fb3_ref[0, 0])
        out_ref[...] = o

    grid_spec = pltpu.PrefetchScalarGridSpec(
        num_scalar_prefetch=1,
        grid=(G,),
        in_specs=[
            pl.BlockSpec((1, 1, 2 * SB), lambda g, sp: (g, 0, 0)),
            pl.BlockSpec((SB, D), lambda g, sp: (g, 0)),
            pl.BlockSpec((3 * D, D), lambda g, sp: (0, 0)),
            pl.BlockSpec((1, D), lambda g, sp: (0, 0)),
            pl.BlockSpec((D, 1), lambda g, sp: (0, 0)),
            pl.BlockSpec((1, 1), lambda g, sp: (0, 0)),
            pl.BlockSpec((2 * D, H1), lambda g, sp: (0, 0)),
            pl.BlockSpec((1, H1), lambda g, sp: (0, 0)),
            pl.BlockSpec((H1, H2), lambda g, sp: (0, 0)),
            pl.BlockSpec((1, H2), lambda g, sp: (0, 0)),
            pl.BlockSpec((H2, 1), lambda g, sp: (0, 0)),
            pl.BlockSpec((1, 1), lambda g, sp: (0, 0)),
            pl.BlockSpec(memory_space=pltpu.MemorySpace.HBM),
        ],
        out_specs=pl.BlockSpec((SB, 1), lambda g, sp: (g, 0)),
        scratch_shapes=[pltpu.VMEM((SB, D + 1), jnp.float32)],
    )
    return pl.pallas_call(
        body,
        grid_spec=grid_spec,
        out_shape=jax.ShapeDtypeStruct((B, 1), jnp.float32),
    )


def kernel(item_id, hist_flat, cu_seqlens, item_table,
           att_W1, att_b1, att_W2, att_b2,
           fc_W1, fc_b1, fc_W2, fc_b2, fc_W3, fc_b3):
    B = item_id.shape[0]
    T = hist_flat.shape[0]
    D = item_table.shape[1]
    H1 = fc_W1.shape[1]
    H2 = fc_W2.shape[1]
    G = B // SB

    hist, tgt = _sc_gather(T, B, D)(hist_flat, item_id, item_table)

    cu = cu_seqlens.astype(jnp.int32)
    cub = jnp.concatenate(
        [
            cu[:-1].reshape(G, SB),
            cu[SB::SB].reshape(G, 1),
            jnp.full((G, SB - 1), jnp.int32(T)),
        ],
        axis=1,
    ).reshape(G, 1, 2 * SB)

    out = _tc_din(T, B, D, H1, H2)(
        cu, cub, tgt,
        att_W1, att_b1.reshape(1, D), att_W2, att_b2.reshape(1, 1),
        fc_W1, fc_b1.reshape(1, H1), fc_W2, fc_b2.reshape(1, H2),
        fc_W3, fc_b3.reshape(1, 1),
        hist,
    )
    return out
